# baseline (speedup 1.0000x reference)
import functools

import jax
import jax.numpy as jnp
from jax import lax
from jax.experimental import pallas as pl
from jax.experimental.pallas import tpu as pltpu
from jax.experimental.pallas import tpu_sc as plsc

_M = 100000
_N = 16384
_D = 128

_NW = 32
_OWN_BIG = 3128
_OWN_SMALL = 3120
_BIG_TILES = 20
_SLOTS = 3136
_L = 16
_NVEC = _N // _L
_SVEC = _SLOTS // _L
_CHUNK = 112
_NFULL = 27
_TAIL_BIG = _OWN_BIG - _NFULL * _CHUNK
_TAIL_SMALL = _OWN_SMALL - _NFULL * _CHUNK
_MAXCH = 28
_COMP = _SLOTS + _L
_TSPAD = _N + 8


def _sc_body(nid_hbm, msgs_hbm, ts_hbm,
             mask_out, ts_out, msgs_out,
             nid_v, lane_buf, mask_v, ts_all_v, ts_o_v,
             comp_pos, comp_glb, glb2d,
             row_a, row_b, sem_a, sem_b, sem_sa, sem_sb, sem_z, sem_in):
    wid = lax.axis_index("s") * 2 + lax.axis_index("c")
    big = wid < _BIG_TILES
    base = wid * _OWN_BIG - jnp.maximum(wid - _BIG_TILES, 0) * 8
    own = jnp.where(big, _OWN_BIG, _OWN_SMALL)

    cp_nid = pltpu.async_copy(nid_hbm, nid_v, sem_in)
    cp_ts = pltpu.async_copy(ts_hbm, ts_all_v, sem_in)

    lanes = lax.iota(jnp.int32, _L)
    lane_off = lanes * _SLOTS
    minus1 = jnp.full((_L,), -1, jnp.int32)
    zerov = jnp.zeros((_L,), jnp.float32)

    with jax.named_scope("zero_buf"):
        def zb_body(i, c):
            for u in range(8):
                row_b[i, pl.ds(u * _L, _L)] = zerov
            return c
        lax.fori_loop(0, _CHUNK, zb_body, 0)

    with jax.named_scope("init_buf"):
        def init_body(i, c):
            for u in range(8):
                lane_buf[pl.ds(i * (8 * _L) + u * _L, _L)] = minus1
            return c
        lax.fori_loop(0, (_L * _SLOTS) // (8 * _L), init_body, 0)
        cp_nid.wait()
        cp_ts.wait()

    with jax.named_scope("zero_fill"):
        zcps = []
        for c in range(_NFULL):
            zcps.append(pltpu.async_copy(
                row_b, msgs_out.at[pl.ds(base + c * _CHUNK, _CHUNK)],
                sem_z))
        toff = _NFULL * _CHUNK

        @pl.when(big)
        def _():
            pltpu.async_copy(
                row_b.at[pl.ds(0, _TAIL_BIG)],
                msgs_out.at[pl.ds(base + toff, _TAIL_BIG)], sem_z).wait()

        @pl.when(jnp.logical_not(big))
        def _():
            pltpu.async_copy(
                row_b.at[pl.ds(0, _TAIL_SMALL)],
                msgs_out.at[pl.ds(base + toff, _TAIL_SMALL)], sem_z).wait()

    with jax.named_scope("scan_msgs"):
        def scan_body(j, c):
            for u in range(4):
                o = j * (4 * _L) + u * _L
                nids = nid_v[pl.ds(o, _L)]
                pos = lanes + o
                loc = nids - base
                inr = (loc >= 0) & (loc < own)
                safe = jnp.where(inr, loc, 0)
                plsc.store_scatter(lane_buf, [lane_off + safe], pos,
                                   mask=inr)
            return c
        lax.fori_loop(0, _NVEC // 4, scan_body, 0)

    with jax.named_scope("combine"):
        def comb_body(v, off):
            o = v * _L
            acc = lane_buf[pl.ds(o, _L)]
            for l in range(1, _L):
                acc = jnp.maximum(acc, lane_buf[pl.ds(l * _SLOTS + o, _L)])
            upd = acc >= 0
            mask_v[pl.ds(o, _L)] = jnp.where(upd, 1, 0).astype(jnp.int32)
            ts_o_v[pl.ds(o, _L)] = plsc.load_gather(
                ts_all_v, [jnp.where(upd, acc, _N)])
            plsc.store_compressed(comp_pos.at[pl.ds(off, _L)], acc,
                                  mask=upd)
            plsc.store_compressed(comp_glb.at[pl.ds(off, _L)],
                                  base + o + lanes, mask=upd)
            return off + plsc.all_reduce_population_count(upd)[0]
        nupd = lax.fori_loop(0, _SVEC, comb_body, 0)

        @pl.when(big)
        def _():
            pltpu.sync_copy(mask_v.at[pl.ds(0, _OWN_BIG)],
                            mask_out.at[pl.ds(base, _OWN_BIG)])
            pltpu.sync_copy(ts_o_v.at[pl.ds(0, _OWN_BIG)],
                            ts_out.at[pl.ds(base, _OWN_BIG)])

        @pl.when(jnp.logical_not(big))
        def _():
            pltpu.sync_copy(mask_v.at[pl.ds(0, _OWN_SMALL)],
                            mask_out.at[pl.ds(base, _OWN_SMALL)])
            pltpu.sync_copy(ts_o_v.at[pl.ds(0, _OWN_SMALL)],
                            ts_out.at[pl.ds(base, _OWN_SMALL)])

    with jax.named_scope("pad_pack"):
        nch = (nupd + (_CHUNK - 1)) // _CHUNK
        npad = nch * _CHUNK
        pos0 = jnp.full((_L,), comp_pos[pl.ds(0, _L)][0], jnp.int32)
        glb0 = jnp.full((_L,), comp_glb[pl.ds(0, _L)][0], jnp.int32)
        o0 = (nupd // _L) * _L
        keep = lanes < (nupd - o0)
        comp_pos[pl.ds(o0, _L)] = jnp.where(
            keep, comp_pos[pl.ds(o0, _L)], pos0)
        comp_glb[pl.ds(o0, _L)] = jnp.where(
            keep, comp_glb[pl.ds(o0, _L)], glb0)

        def pad_body(i, c):
            comp_pos[pl.ds(i * _L, _L)] = pos0
            comp_glb[pl.ds(i * _L, _L)] = glb0
            return c
        lax.fori_loop(o0 // _L + 1, nch * (_CHUNK // _L), pad_body, 0)

        def pack_body(r, c):
            for k in range(_CHUNK // _L):
                glb2d[r, pl.ds(k * _L, _L)] = \
                    comp_glb[pl.ds(r * _CHUNK + k * _L, _L)]
            return c
        lax.fori_loop(0, nch, pack_body, 0)

        for cp in zcps:
            cp.wait()

    with jax.named_scope("row_gather"):
        def drain(sem):
            pltpu.make_async_copy(
                msgs_hbm.at[pl.ds(0, _CHUNK)], row_a, sem).wait()

        def pair_body(p, c):
            c0 = 2 * p
            c1 = 2 * p + 1

            @pl.when(p > 0)
            def _():
                drain(sem_sa)

            g0 = pltpu.async_copy(
                msgs_hbm.at[comp_pos.at[pl.ds(c0 * _CHUNK, _CHUNK)]],
                row_a, sem_a)

            @pl.when((p > 0) & (c1 < nch))
            def _():
                drain(sem_sb)

            @pl.when(c1 < nch)
            def _():
                g1 = pltpu.async_copy(
                    msgs_hbm.at[comp_pos.at[pl.ds(c1 * _CHUNK, _CHUNK)]],
                    row_b, sem_b)
                g0.wait()
                pltpu.async_copy(
                    row_a, msgs_out.at[glb2d.at[c0]], sem_sa)
                g1.wait()
                pltpu.async_copy(
                    row_b, msgs_out.at[glb2d.at[c1]], sem_sb)

            @pl.when(c1 >= nch)
            def _():
                g0.wait()
                pltpu.async_copy(
                    row_a, msgs_out.at[glb2d.at[c0]], sem_sa)
            return c
        lax.fori_loop(0, (nch + 1) // 2, pair_body, 0)

        @pl.when(nch > 0)
        def _():
            drain(sem_sa)

        @pl.when(nch > 1)
        def _():
            drain(sem_sb)


@jax.jit
def _run(nid, msgs, ts_p):
    mesh = plsc.VectorSubcoreMesh(core_axis_name="c", subcore_axis_name="s")
    f = pl.kernel(
        _sc_body,
        out_type=[
            jax.ShapeDtypeStruct((_M,), jnp.int32),
            jax.ShapeDtypeStruct((_M,), jnp.float32),
            jax.ShapeDtypeStruct((_M, _D), jnp.float32),
        ],
        mesh=mesh,
        compiler_params=pltpu.CompilerParams(needs_layout_passes=False),
        scratch_types=[
            pltpu.VMEM((_N,), jnp.int32),
            pltpu.VMEM((_L * _SLOTS,), jnp.int32),
            pltpu.VMEM((_SLOTS,), jnp.int32),
            pltpu.VMEM((_TSPAD,), jnp.float32),
            pltpu.VMEM((_SLOTS,), jnp.float32),
            pltpu.VMEM((_COMP,), jnp.int32),
            pltpu.VMEM((_COMP,), jnp.int32),
            pltpu.VMEM((_MAXCH, _CHUNK), jnp.int32),
            pltpu.VMEM((_CHUNK, _D), jnp.float32),
            pltpu.VMEM((_CHUNK, _D), jnp.float32),
            pltpu.SemaphoreType.DMA,
            pltpu.SemaphoreType.DMA,
            pltpu.SemaphoreType.DMA,
            pltpu.SemaphoreType.DMA,
            pltpu.SemaphoreType.DMA,
            pltpu.SemaphoreType.DMA,
        ],
    )
    return f(nid, msgs, ts_p)


def kernel(node_ids, msgs, ts):
    nid = node_ids.astype(jnp.int32)
    ts_p = jnp.concatenate([ts, jnp.zeros((_TSPAD - _N,), ts.dtype)], axis=0)
    mask_i, ts_o, msgs_o = _run(nid, msgs, ts_p)
    return (mask_i.astype(bool), msgs_o, ts_o)


# speedup vs baseline: 1.0140x; 1.0140x over previous
import functools

import jax
import jax.numpy as jnp
from jax import lax
from jax.experimental import pallas as pl
from jax.experimental.pallas import tpu as pltpu
from jax.experimental.pallas import tpu_sc as plsc

_M = 100000
_N = 16384
_D = 128

_NW = 32
_OWN_BIG = 3128
_OWN_SMALL = 3120
_BIG_TILES = 20
_SLOTS = 3136
_L = 16
_NVEC = _N // _L
_SVEC = _SLOTS // _L
_CHUNK = 112
_NFULL = 27
_TAIL_BIG = _OWN_BIG - _NFULL * _CHUNK
_TAIL_SMALL = _OWN_SMALL - _NFULL * _CHUNK
_MAXCH = 28
_COMP = _SLOTS + _L
_TSPAD = _N + 8


def _sc_body(nid_hbm, msgs_hbm, ts_hbm,
             mask_out, ts_out, msgs_out,
             nid_v, lane_buf, mask_v, ts_all_v, ts_o_v,
             comp_pos, comp_glb, glb2d,
             row_a, row_b, sem_a, sem_b, sem_z, sem_in):
    wid = lax.axis_index("s") * 2 + lax.axis_index("c")
    big = wid < _BIG_TILES
    base = wid * _OWN_BIG - jnp.maximum(wid - _BIG_TILES, 0) * 8
    own = jnp.where(big, _OWN_BIG, _OWN_SMALL)

    cp_nid = pltpu.async_copy(nid_hbm, nid_v, sem_in)
    cp_ts = pltpu.async_copy(ts_hbm, ts_all_v, sem_in)

    lanes = lax.iota(jnp.int32, _L)
    lane_off = lanes * _SLOTS
    minus1 = jnp.full((_L,), -1, jnp.int32)
    zerov = jnp.zeros((_L,), jnp.float32)

    with jax.named_scope("zero_buf"):
        def zb_body(i, c):
            for u in range(8):
                row_b[i, pl.ds(u * _L, _L)] = zerov
            return c
        lax.fori_loop(0, _CHUNK, zb_body, 0)

    with jax.named_scope("init_buf"):
        def init_body(i, c):
            for u in range(8):
                lane_buf[pl.ds(i * (8 * _L) + u * _L, _L)] = minus1
            return c
        lax.fori_loop(0, (_L * _SLOTS) // (8 * _L), init_body, 0)
        cp_nid.wait()
        cp_ts.wait()

    with jax.named_scope("zero_fill"):
        zcps = []
        for c in range(_NFULL):
            zcps.append(pltpu.async_copy(
                row_b, msgs_out.at[pl.ds(base + c * _CHUNK, _CHUNK)],
                sem_z))
        toff = _NFULL * _CHUNK

        @pl.when(big)
        def _():
            pltpu.async_copy(
                row_b.at[pl.ds(0, _TAIL_BIG)],
                msgs_out.at[pl.ds(base + toff, _TAIL_BIG)], sem_z).wait()

        @pl.when(jnp.logical_not(big))
        def _():
            pltpu.async_copy(
                row_b.at[pl.ds(0, _TAIL_SMALL)],
                msgs_out.at[pl.ds(base + toff, _TAIL_SMALL)], sem_z).wait()

    with jax.named_scope("scan_msgs"):
        def scan_body(j, c):
            for u in range(4):
                o = j * (4 * _L) + u * _L
                nids = nid_v[pl.ds(o, _L)]
                pos = lanes + o
                loc = nids - base
                inr = (loc >= 0) & (loc < own)
                safe = jnp.where(inr, loc, 0)
                plsc.store_scatter(lane_buf, [lane_off + safe], pos,
                                   mask=inr)
            return c
        lax.fori_loop(0, _NVEC // 4, scan_body, 0)

    with jax.named_scope("combine"):
        def comb_body(v, off):
            o = v * _L
            acc = lane_buf[pl.ds(o, _L)]
            for l in range(1, _L):
                acc = jnp.maximum(acc, lane_buf[pl.ds(l * _SLOTS + o, _L)])
            upd = acc >= 0
            mask_v[pl.ds(o, _L)] = jnp.where(upd, 1, 0).astype(jnp.int32)
            ts_o_v[pl.ds(o, _L)] = plsc.load_gather(
                ts_all_v, [jnp.where(upd, acc, _N)])
            plsc.store_compressed(comp_pos.at[pl.ds(off, _L)], acc,
                                  mask=upd)
            plsc.store_compressed(comp_glb.at[pl.ds(off, _L)],
                                  base + o + lanes, mask=upd)
            return off + plsc.all_reduce_population_count(upd)[0]
        nupd = lax.fori_loop(0, _SVEC, comb_body, 0)

        @pl.when(big)
        def _():
            pltpu.sync_copy(mask_v.at[pl.ds(0, _OWN_BIG)],
                            mask_out.at[pl.ds(base, _OWN_BIG)])
            pltpu.sync_copy(ts_o_v.at[pl.ds(0, _OWN_BIG)],
                            ts_out.at[pl.ds(base, _OWN_BIG)])

        @pl.when(jnp.logical_not(big))
        def _():
            pltpu.sync_copy(mask_v.at[pl.ds(0, _OWN_SMALL)],
                            mask_out.at[pl.ds(base, _OWN_SMALL)])
            pltpu.sync_copy(ts_o_v.at[pl.ds(0, _OWN_SMALL)],
                            ts_out.at[pl.ds(base, _OWN_SMALL)])

    with jax.named_scope("pad_pack"):
        nch = (nupd + (_CHUNK - 1)) // _CHUNK
        npad = nch * _CHUNK
        pos0 = jnp.full((_L,), comp_pos[pl.ds(0, _L)][0], jnp.int32)
        glb0 = jnp.full((_L,), comp_glb[pl.ds(0, _L)][0], jnp.int32)
        o0 = (nupd // _L) * _L
        keep = lanes < (nupd - o0)
        comp_pos[pl.ds(o0, _L)] = jnp.where(
            keep, comp_pos[pl.ds(o0, _L)], pos0)
        comp_glb[pl.ds(o0, _L)] = jnp.where(
            keep, comp_glb[pl.ds(o0, _L)], glb0)

        def pad_body(i, c):
            comp_pos[pl.ds(i * _L, _L)] = pos0
            comp_glb[pl.ds(i * _L, _L)] = glb0
            return c
        lax.fori_loop(o0 // _L + 1, nch * (_CHUNK // _L), pad_body, 0)

        def pack_body(r, c):
            for k in range(_CHUNK // _L):
                glb2d[r, pl.ds(k * _L, _L)] = \
                    comp_glb[pl.ds(r * _CHUNK + k * _L, _L)]
            return c
        lax.fori_loop(0, nch, pack_body, 0)

        for cp in zcps:
            cp.wait()

    with jax.named_scope("row_gather"):
        def pair_body(p, c):
            c0 = 2 * p
            c1 = 2 * p + 1
            g0 = pltpu.async_copy(
                msgs_hbm.at[comp_pos.at[pl.ds(c0 * _CHUNK, _CHUNK)]],
                row_a, sem_a)

            @pl.when(c1 < nch)
            def _():
                g1 = pltpu.async_copy(
                    msgs_hbm.at[comp_pos.at[pl.ds(c1 * _CHUNK, _CHUNK)]],
                    row_b, sem_b)
                g0.wait()
                pltpu.async_copy(
                    row_a, msgs_out.at[glb2d.at[c0]], sem_a).wait()
                g1.wait()
                pltpu.async_copy(
                    row_b, msgs_out.at[glb2d.at[c1]], sem_b).wait()

            @pl.when(c1 >= nch)
            def _():
                g0.wait()
                pltpu.async_copy(
                    row_a, msgs_out.at[glb2d.at[c0]], sem_a).wait()
            return c
        lax.fori_loop(0, (nch + 1) // 2, pair_body, 0)


@jax.jit
def _run(nid, msgs, ts_p):
    mesh = plsc.VectorSubcoreMesh(core_axis_name="c", subcore_axis_name="s")
    f = pl.kernel(
        _sc_body,
        out_type=[
            jax.ShapeDtypeStruct((_M,), jnp.int32),
            jax.ShapeDtypeStruct((_M,), jnp.float32),
            jax.ShapeDtypeStruct((_M, _D), jnp.float32),
        ],
        mesh=mesh,
        compiler_params=pltpu.CompilerParams(needs_layout_passes=False),
        scratch_types=[
            pltpu.VMEM((_N,), jnp.int32),
            pltpu.VMEM((_L * _SLOTS,), jnp.int32),
            pltpu.VMEM((_SLOTS,), jnp.int32),
            pltpu.VMEM((_TSPAD,), jnp.float32),
            pltpu.VMEM((_SLOTS,), jnp.float32),
            pltpu.VMEM((_COMP,), jnp.int32),
            pltpu.VMEM((_COMP,), jnp.int32),
            pltpu.VMEM((_MAXCH, _CHUNK), jnp.int32),
            pltpu.VMEM((_CHUNK, _D), jnp.float32),
            pltpu.VMEM((_CHUNK, _D), jnp.float32),
            pltpu.SemaphoreType.DMA,
            pltpu.SemaphoreType.DMA,
            pltpu.SemaphoreType.DMA,
            pltpu.SemaphoreType.DMA,
        ],
    )
    return f(nid, msgs, ts_p)


def kernel(node_ids, msgs, ts):
    nid = node_ids.astype(jnp.int32)
    ts_p = jnp.concatenate([ts, jnp.zeros((_TSPAD - _N,), ts.dtype)], axis=0)
    mask_i, ts_o, msgs_o = _run(nid, msgs, ts_p)
    return (mask_i.astype(bool), msgs_o, ts_o)
